# baseline (speedup 1.0000x reference)
import functools

import jax
import jax.numpy as jnp
from jax import lax
from jax.experimental import pallas as pl
from jax.experimental.pallas import tpu as pltpu
from jax.experimental.pallas import tpu_sc as plsc

N = 10000
E = 320000
D = 128

NT = 10112
TILES = 32
CH = 80
C = 128
EP = TILES * CH * C
RPT = NT // 16

_MESH = dict(core_axis_name="c", subcore_axis_name="s", num_cores=2,
             num_subcores=16)
_SC_PARAMS = pltpu.CompilerParams(needs_layout_passes=False)


def _tile_id():
    return lax.axis_index("c") * 16 + lax.axis_index("s")


def _deg_body(src_hbm, dst_hbm, zeros_hbm, out_do, out_di,
              src_v, dst_v, ho_v, hi_v):
    t = _tile_id()
    pltpu.sync_copy(src_hbm.at[t], src_v)
    pltpu.sync_copy(dst_hbm.at[t], dst_v)
    pltpu.sync_copy(zeros_hbm, ho_v)
    pltpu.sync_copy(zeros_hbm, hi_v)
    ones = jnp.ones((16,), jnp.float32)

    def grp(i, _):
        j = i // 8
        m = (i % 8) * 16
        plsc.addupdate_scatter(ho_v, [src_v[j, pl.ds(m, 16)]], ones)
        plsc.addupdate_scatter(hi_v, [dst_v[j, pl.ds(m, 16)]], ones)
        return 0
    lax.fori_loop(0, CH * 8, grp, 0)
    pltpu.sync_copy(ho_v, out_do.at[t])
    pltpu.sync_copy(hi_v, out_di.at[t])


_deg_kernel = functools.partial(
    pl.kernel,
    out_type=[jax.ShapeDtypeStruct((TILES, NT), jnp.float32),
              jax.ShapeDtypeStruct((TILES, NT), jnp.float32)],
    mesh=plsc.VectorSubcoreMesh(**_MESH),
    compiler_params=_SC_PARAMS,
    scratch_types=[
        pltpu.VMEM((CH, C), jnp.int32),
        pltpu.VMEM((CH, C), jnp.int32),
        pltpu.VMEM((NT,), jnp.float32),
        pltpu.VMEM((NT,), jnp.float32),
    ],
)(_deg_body)


def _norm_body(do_ref, di_ref, out_ref):
    deg = jnp.stack([jnp.sum(do_ref[...], axis=0),
                     jnp.sum(di_ref[...], axis=0)])
    out_ref[...] = lax.rsqrt(jnp.maximum(deg, 1.0))


def _norm_kernel(degp_do, degp_di):
    return pl.pallas_call(
        _norm_body,
        out_shape=jax.ShapeDtypeStruct((2, NT), jnp.float32),
    )(degp_do, degp_di)


def _ew2_body(norms_hbm, src_hbm, dst_hbm, ew_hbm, out_hbm,
              no_v, ni_v, src_v, dst_v, ew_v):
    t = _tile_id()
    pltpu.sync_copy(norms_hbm.at[0], no_v)
    pltpu.sync_copy(norms_hbm.at[1], ni_v)
    pltpu.sync_copy(src_hbm.at[t], src_v)
    pltpu.sync_copy(dst_hbm.at[t], dst_v)
    pltpu.sync_copy(ew_hbm.at[t], ew_v)

    def group(i, _):
        j = i // 8
        m = (i % 8) * 16
        sidx = src_v[j, pl.ds(m, 16)]
        didx = dst_v[j, pl.ds(m, 16)]
        a = plsc.load_gather(no_v, [sidx])
        b = plsc.load_gather(ni_v, [didx])
        ew_v[j, pl.ds(m, 16)] = ew_v[j, pl.ds(m, 16)] * a * b
        return 0
    lax.fori_loop(0, CH * 8, group, 0)
    pltpu.sync_copy(ew_v, out_hbm.at[t])


_ew2_kernel = functools.partial(
    pl.kernel,
    out_type=jax.ShapeDtypeStruct((TILES, CH, C), jnp.float32),
    mesh=plsc.VectorSubcoreMesh(**_MESH),
    compiler_params=_SC_PARAMS,
    scratch_types=[
        pltpu.VMEM((NT,), jnp.float32),
        pltpu.VMEM((NT,), jnp.float32),
        pltpu.VMEM((CH, C), jnp.int32),
        pltpu.VMEM((CH, C), jnp.int32),
        pltpu.VMEM((CH, C), jnp.float32),
    ],
)(_ew2_body)


NBUF = 2
CH0 = 48
CH1 = (2 * CH) - CH0
CHMAX = max(CH0, CH1)
TOTCH = TILES * CH


def _spmm_body(h_hbm, src_hbm, dw_hbm, zeros_hbm, out_hbm,
               src_v, dw_v, rows, dwsems, gsems, ssems, sh_agg):
    c = lax.axis_index("c")
    s = lax.axis_index("s")
    nch = jnp.where(c == 0, CH0, CH1)
    base = jnp.where(c == 0, s * CH0, 16 * CH0 + s * CH1)
    nrnd = nch // NBUF
    pltpu.sync_copy(src_hbm.at[pl.ds(base, CHMAX)], src_v)
    pltpu.sync_copy(zeros_hbm.at[pl.ds(s * RPT, RPT)],
                    sh_agg.at[pl.ds(s * RPT, RPT)])

    for b in range(NBUF):
        pltpu.async_copy(dw_hbm.at[base + b], dw_v.at[b], dwsems.at[b])
        pltpu.async_copy(h_hbm.at[src_v.at[b]], rows.at[b], gsems.at[b])
    plsc.subcore_barrier()

    def scale(b):
        def edge(i, _):
            wi = plsc.load_gather(
                dw_v, [jnp.full((16,), b, jnp.int32),
                       jnp.full((16,), 1, jnp.int32),
                       jnp.full((16,), i, jnp.int32)])
            w = plsc.bitcast(wi, jnp.float32)
            for k in range(8):
                rows[b, i, pl.ds(k * 16, 16)] = rows[b, i, pl.ds(k * 16, 16)] * w
            return 0
        lax.fori_loop(0, C, edge, 0, unroll=4)

    def rnd(r, _):
        for b in range(NBUF):
            j = r * NBUF + b
            pltpu.make_async_copy(h_hbm.at[src_v.at[j]], rows.at[b],
                                  gsems.at[b]).wait()
            pltpu.make_async_copy(dw_hbm.at[base + j], dw_v.at[b],
                                  dwsems.at[b]).wait()
            scale(b)
            pltpu.async_copy(rows.at[b], sh_agg.at[dw_v.at[b, 0]],
                             ssems.at[b], add=True)
        for b in range(NBUF):
            j = r * NBUF + b

            @pl.when(r < nrnd - 1)
            def _():
                pltpu.make_async_copy(rows.at[b], sh_agg.at[dw_v.at[b, 0]],
                                      ssems.at[b]).wait()
                pltpu.async_copy(dw_hbm.at[base + j + NBUF], dw_v.at[b],
                                 dwsems.at[b])
                pltpu.async_copy(h_hbm.at[src_v.at[j + NBUF]], rows.at[b],
                                 gsems.at[b])
        return 0
    lax.fori_loop(0, nrnd, rnd, 0)
    for b in range(NBUF):
        pltpu.make_async_copy(rows.at[b], sh_agg.at[dw_v.at[b, 0]],
                              ssems.at[b]).wait()
    plsc.subcore_barrier()
    pltpu.sync_copy(sh_agg.at[pl.ds(s * RPT, RPT)],
                    out_hbm.at[c, pl.ds(s * RPT, RPT)])


_spmm_kernel = functools.partial(
    pl.kernel,
    out_type=jax.ShapeDtypeStruct((2, NT, D), jnp.float32),
    mesh=plsc.VectorSubcoreMesh(**_MESH),
    compiler_params=_SC_PARAMS,
    scratch_types=[
        pltpu.VMEM((CHMAX, C), jnp.int32),
        pltpu.VMEM((NBUF, 2, C), jnp.int32),
        pltpu.VMEM((NBUF, C, D), jnp.float32),
        pltpu.SemaphoreType.DMA((NBUF,)),
        pltpu.SemaphoreType.DMA((NBUF,)),
        pltpu.SemaphoreType.DMA((NBUF,)),
        pltpu.VMEM_SHARED((NT, D), jnp.float32),
    ],
)(_spmm_body)


_RB = 2528


def _mm_body(x_ref, w_ref, o_ref):
    o_ref[...] = jnp.dot(x_ref[...], w_ref[...],
                         preferred_element_type=jnp.float32)


def _mm_kernel(x, w):
    return pl.pallas_call(
        _mm_body,
        grid=(NT // _RB,),
        in_specs=[
            pl.BlockSpec((_RB, D), lambda i: (i, 0)),
            pl.BlockSpec((D, D), lambda i: (0, 0)),
        ],
        out_specs=pl.BlockSpec((_RB, D), lambda i: (i, 0)),
        out_shape=jax.ShapeDtypeStruct((NT, D), jnp.float32),
    )(x, w)


def _postmm_body(p_ref, b_ref, w_ref, o_ref):
    y = jax.nn.relu(p_ref[0] + p_ref[1] + b_ref[...][None, :])
    o_ref[...] = jnp.dot(y, w_ref[...], preferred_element_type=jnp.float32)


def _postmm_kernel(p, b, w):
    return pl.pallas_call(
        _postmm_body,
        grid=(NT // _RB,),
        in_specs=[
            pl.BlockSpec((2, _RB, D), lambda i: (0, i, 0)),
            pl.BlockSpec((D,), lambda i: (0,)),
            pl.BlockSpec((D, D), lambda i: (0, 0)),
        ],
        out_specs=pl.BlockSpec((_RB, D), lambda i: (i, 0)),
        out_shape=jax.ShapeDtypeStruct((NT, D), jnp.float32),
    )(p, b, w)


def _final_body(p_ref, b_ref, wd1_ref, bd1_ref, wd2_ref, bd2_ref,
                wd3_ref, bd3_ref, o_ref):
    y = jax.nn.relu(p_ref[0, :N, :] + p_ref[1, :N, :] + b_ref[...][None, :])
    hg = jnp.sum(y, axis=0, keepdims=True) * (1.0 / N)
    hg = jax.nn.relu(jnp.dot(hg, wd1_ref[...],
                             preferred_element_type=jnp.float32)
                     + bd1_ref[...][None, :])
    hg = jax.nn.relu(jnp.dot(hg, wd2_ref[...],
                             preferred_element_type=jnp.float32)
                     + bd2_ref[...][None, :])
    hg = (jnp.dot(hg, wd3_ref[...], preferred_element_type=jnp.float32)
          + bd3_ref[...][None, :])
    o_ref[...] = jax.nn.sigmoid(hg)


def _final_kernel(p, b3, wd1, bd1, wd2, bd2, wd3, bd3):
    return pl.pallas_call(
        _final_body,
        out_shape=jax.ShapeDtypeStruct((1, 1), jnp.float32),
    )(p, b3, wd1, bd1, wd2, bd2, wd3, bd3)


def kernel(in_feat, edge_index, e_weight, W1, b1, W2, b2, W3, b3,
           Wd1, bd1, Wd2, bd2, Wd3, bd3):
    pad = EP - E
    src3 = jnp.pad(edge_index[0].astype(jnp.int32), (0, pad),
                   constant_values=N).reshape(TILES, CH, C)
    dst3 = jnp.pad(edge_index[1].astype(jnp.int32), (0, pad),
                   constant_values=N).reshape(TILES, CH, C)
    ew3 = jnp.pad(e_weight, (0, pad)).reshape(TILES, CH, C)
    x_pad = jnp.pad(in_feat, ((0, NT - N), (0, 0)))
    zeros_nt = jnp.zeros((NT, D), jnp.float32)

    zeros_1nt = jnp.zeros((NT,), jnp.float32)
    degp_do, degp_di = _deg_kernel(src3, dst3, zeros_1nt)
    norms = _norm_kernel(degp_do, degp_di)
    ew2 = _ew2_kernel(norms, src3, dst3, ew3)

    src_f = src3.reshape(TOTCH, C)
    dw_f = jnp.concatenate(
        [dst3.reshape(TOTCH, 1, C),
         jax.lax.bitcast_convert_type(ew2, jnp.int32).reshape(TOTCH, 1, C)],
        axis=1)

    t1 = _mm_kernel(x_pad, W1)
    p1 = _spmm_kernel(t1, src_f, dw_f, zeros_nt)
    t2 = _postmm_kernel(p1, b1, W2)
    p2 = _spmm_kernel(t2, src_f, dw_f, zeros_nt)
    t3 = _postmm_kernel(p2, b2, W3)
    p3 = _spmm_kernel(t3, src_f, dw_f, zeros_nt)
    return _final_kernel(p3, b3, Wd1, bd1, Wd2, bd2, Wd3, bd3)


# speedup vs baseline: 1.2055x; 1.2055x over previous
import functools

import jax
import jax.numpy as jnp
from jax import lax
from jax.experimental import pallas as pl
from jax.experimental.pallas import tpu as pltpu
from jax.experimental.pallas import tpu_sc as plsc

N = 10000
E = 320000
D = 128

NT = 10112
TILES = 32
CH = 80
C = 128
EP = TILES * CH * C
RPT = NT // 16

_MESH = dict(core_axis_name="c", subcore_axis_name="s", num_cores=2,
             num_subcores=16)
_SC_PARAMS = pltpu.CompilerParams(needs_layout_passes=False)


def _tile_id():
    return lax.axis_index("c") * 16 + lax.axis_index("s")


def _deg_body(src_hbm, dst_hbm, zeros_hbm, out_do, out_di,
              src_v, dst_v, ho_v, hi_v):
    t = _tile_id()
    pltpu.sync_copy(src_hbm.at[t], src_v)
    pltpu.sync_copy(dst_hbm.at[t], dst_v)
    pltpu.sync_copy(zeros_hbm, ho_v)
    pltpu.sync_copy(zeros_hbm, hi_v)
    ones = jnp.ones((16,), jnp.float32)

    def grp(i, _):
        j = i // 8
        m = (i % 8) * 16
        plsc.addupdate_scatter(ho_v, [src_v[j, pl.ds(m, 16)]], ones)
        plsc.addupdate_scatter(hi_v, [dst_v[j, pl.ds(m, 16)]], ones)
        return 0
    lax.fori_loop(0, CH * 8, grp, 0)
    pltpu.sync_copy(ho_v, out_do.at[t])
    pltpu.sync_copy(hi_v, out_di.at[t])


_deg_kernel = functools.partial(
    pl.kernel,
    out_type=[jax.ShapeDtypeStruct((TILES, NT), jnp.float32),
              jax.ShapeDtypeStruct((TILES, NT), jnp.float32)],
    mesh=plsc.VectorSubcoreMesh(**_MESH),
    compiler_params=_SC_PARAMS,
    scratch_types=[
        pltpu.VMEM((CH, C), jnp.int32),
        pltpu.VMEM((CH, C), jnp.int32),
        pltpu.VMEM((NT,), jnp.float32),
        pltpu.VMEM((NT,), jnp.float32),
    ],
)(_deg_body)


def _norm_body(do_ref, di_ref, out_ref):
    deg = jnp.stack([jnp.sum(do_ref[...], axis=0),
                     jnp.sum(di_ref[...], axis=0)])
    out_ref[...] = lax.rsqrt(jnp.maximum(deg, 1.0))


def _norm_kernel(degp_do, degp_di):
    return pl.pallas_call(
        _norm_body,
        out_shape=jax.ShapeDtypeStruct((2, NT), jnp.float32),
    )(degp_do, degp_di)


def _ew2_body(norms_hbm, src_hbm, dst_hbm, ew_hbm, out_hbm,
              no_v, ni_v, src_v, dst_v, ew_v):
    t = _tile_id()
    pltpu.sync_copy(norms_hbm.at[0], no_v)
    pltpu.sync_copy(norms_hbm.at[1], ni_v)
    pltpu.sync_copy(src_hbm.at[t], src_v)
    pltpu.sync_copy(dst_hbm.at[t], dst_v)
    pltpu.sync_copy(ew_hbm.at[t], ew_v)

    def group(i, _):
        j = i // 8
        m = (i % 8) * 16
        sidx = src_v[j, pl.ds(m, 16)]
        didx = dst_v[j, pl.ds(m, 16)]
        a = plsc.load_gather(no_v, [sidx])
        b = plsc.load_gather(ni_v, [didx])
        ew_v[j, pl.ds(m, 16)] = ew_v[j, pl.ds(m, 16)] * a * b
        return 0
    lax.fori_loop(0, CH * 8, group, 0)
    pltpu.sync_copy(ew_v, out_hbm.at[t])


_ew2_kernel = functools.partial(
    pl.kernel,
    out_type=jax.ShapeDtypeStruct((TILES, CH, C), jnp.float32),
    mesh=plsc.VectorSubcoreMesh(**_MESH),
    compiler_params=_SC_PARAMS,
    scratch_types=[
        pltpu.VMEM((NT,), jnp.float32),
        pltpu.VMEM((NT,), jnp.float32),
        pltpu.VMEM((CH, C), jnp.int32),
        pltpu.VMEM((CH, C), jnp.int32),
        pltpu.VMEM((CH, C), jnp.float32),
    ],
)(_ew2_body)


NBUF = 2
CH0 = 112
CH1 = (2 * CH) - CH0
CHMAX = max(CH0, CH1)
TOTCH = TILES * CH


def _spmm_body(h_hbm, src_hbm, dw_hbm, zeros_hbm, out_hbm,
               src_v, dw_v, rows, dwsems, gsems, ssems, sh_agg):
    c = lax.axis_index("c")
    s = lax.axis_index("s")
    nch = jnp.where(c == 0, CH0, CH1)
    base = jnp.where(c == 0, s * CH0, 16 * CH0 + s * CH1)
    nrnd = nch // NBUF
    pltpu.sync_copy(src_hbm.at[pl.ds(base, CHMAX)], src_v)
    pltpu.sync_copy(zeros_hbm.at[pl.ds(s * RPT, RPT)],
                    sh_agg.at[pl.ds(s * RPT, RPT)])

    for b in range(NBUF):
        pltpu.async_copy(dw_hbm.at[base + b], dw_v.at[b], dwsems.at[b])
        pltpu.async_copy(h_hbm.at[src_v.at[b]], rows.at[b], gsems.at[b])
    plsc.subcore_barrier()

    def scale(b):
        def edge(i, _):
            wi = plsc.load_gather(
                dw_v, [jnp.full((16,), b, jnp.int32),
                       jnp.full((16,), 1, jnp.int32),
                       jnp.full((16,), i, jnp.int32)])
            w = plsc.bitcast(wi, jnp.float32)
            for k in range(8):
                rows[b, i, pl.ds(k * 16, 16)] = rows[b, i, pl.ds(k * 16, 16)] * w
            return 0
        lax.fori_loop(0, C, edge, 0, unroll=4)

    def rnd(r, _):
        for b in range(NBUF):
            j = r * NBUF + b
            pltpu.make_async_copy(h_hbm.at[src_v.at[j]], rows.at[b],
                                  gsems.at[b]).wait()
            pltpu.make_async_copy(dw_hbm.at[base + j], dw_v.at[b],
                                  dwsems.at[b]).wait()
            scale(b)
            pltpu.async_copy(rows.at[b], sh_agg.at[dw_v.at[b, 0]],
                             ssems.at[b], add=True)
        for b in range(NBUF):
            j = r * NBUF + b

            @pl.when(r < nrnd - 1)
            def _():
                pltpu.make_async_copy(rows.at[b], sh_agg.at[dw_v.at[b, 0]],
                                      ssems.at[b]).wait()
                pltpu.async_copy(dw_hbm.at[base + j + NBUF], dw_v.at[b],
                                 dwsems.at[b])
                pltpu.async_copy(h_hbm.at[src_v.at[j + NBUF]], rows.at[b],
                                 gsems.at[b])
        return 0
    lax.fori_loop(0, nrnd, rnd, 0)
    for b in range(NBUF):
        pltpu.make_async_copy(rows.at[b], sh_agg.at[dw_v.at[b, 0]],
                              ssems.at[b]).wait()
    plsc.subcore_barrier()
    pltpu.sync_copy(sh_agg.at[pl.ds(s * RPT, RPT)],
                    out_hbm.at[c, pl.ds(s * RPT, RPT)])


_spmm_kernel = functools.partial(
    pl.kernel,
    out_type=jax.ShapeDtypeStruct((2, NT, D), jnp.float32),
    mesh=plsc.VectorSubcoreMesh(**_MESH),
    compiler_params=_SC_PARAMS,
    scratch_types=[
        pltpu.VMEM((CHMAX, C), jnp.int32),
        pltpu.VMEM((NBUF, 2, C), jnp.int32),
        pltpu.VMEM((NBUF, C, D), jnp.float32),
        pltpu.SemaphoreType.DMA((NBUF,)),
        pltpu.SemaphoreType.DMA((NBUF,)),
        pltpu.SemaphoreType.DMA((NBUF,)),
        pltpu.VMEM_SHARED((NT, D), jnp.float32),
    ],
)(_spmm_body)


_RB = 2528


def _mm_body(x_ref, w_ref, o_ref):
    o_ref[...] = jnp.dot(x_ref[...], w_ref[...],
                         preferred_element_type=jnp.float32)


def _mm_kernel(x, w):
    return pl.pallas_call(
        _mm_body,
        grid=(NT // _RB,),
        in_specs=[
            pl.BlockSpec((_RB, D), lambda i: (i, 0)),
            pl.BlockSpec((D, D), lambda i: (0, 0)),
        ],
        out_specs=pl.BlockSpec((_RB, D), lambda i: (i, 0)),
        out_shape=jax.ShapeDtypeStruct((NT, D), jnp.float32),
    )(x, w)


def _postmm_body(p_ref, b_ref, w_ref, o_ref):
    y = jax.nn.relu(p_ref[0] + p_ref[1] + b_ref[...][None, :])
    o_ref[...] = jnp.dot(y, w_ref[...], preferred_element_type=jnp.float32)


def _postmm_kernel(p, b, w):
    return pl.pallas_call(
        _postmm_body,
        grid=(NT // _RB,),
        in_specs=[
            pl.BlockSpec((2, _RB, D), lambda i: (0, i, 0)),
            pl.BlockSpec((D,), lambda i: (0,)),
            pl.BlockSpec((D, D), lambda i: (0, 0)),
        ],
        out_specs=pl.BlockSpec((_RB, D), lambda i: (i, 0)),
        out_shape=jax.ShapeDtypeStruct((NT, D), jnp.float32),
    )(p, b, w)


def _final_body(p_ref, b_ref, wd1_ref, bd1_ref, wd2_ref, bd2_ref,
                wd3_ref, bd3_ref, o_ref):
    y = jax.nn.relu(p_ref[0, :N, :] + p_ref[1, :N, :] + b_ref[...][None, :])
    hg = jnp.sum(y, axis=0, keepdims=True) * (1.0 / N)
    hg = jax.nn.relu(jnp.dot(hg, wd1_ref[...],
                             preferred_element_type=jnp.float32)
                     + bd1_ref[...][None, :])
    hg = jax.nn.relu(jnp.dot(hg, wd2_ref[...],
                             preferred_element_type=jnp.float32)
                     + bd2_ref[...][None, :])
    hg = (jnp.dot(hg, wd3_ref[...], preferred_element_type=jnp.float32)
          + bd3_ref[...][None, :])
    o_ref[...] = jax.nn.sigmoid(hg)


def _final_kernel(p, b3, wd1, bd1, wd2, bd2, wd3, bd3):
    return pl.pallas_call(
        _final_body,
        out_shape=jax.ShapeDtypeStruct((1, 1), jnp.float32),
    )(p, b3, wd1, bd1, wd2, bd2, wd3, bd3)


def kernel(in_feat, edge_index, e_weight, W1, b1, W2, b2, W3, b3,
           Wd1, bd1, Wd2, bd2, Wd3, bd3):
    pad = EP - E
    src3 = jnp.pad(edge_index[0].astype(jnp.int32), (0, pad),
                   constant_values=N).reshape(TILES, CH, C)
    dst3 = jnp.pad(edge_index[1].astype(jnp.int32), (0, pad),
                   constant_values=N).reshape(TILES, CH, C)
    ew3 = jnp.pad(e_weight, (0, pad)).reshape(TILES, CH, C)
    x_pad = jnp.pad(in_feat, ((0, NT - N), (0, 0)))
    zeros_nt = jnp.zeros((NT, D), jnp.float32)

    zeros_1nt = jnp.zeros((NT,), jnp.float32)
    degp_do, degp_di = _deg_kernel(src3, dst3, zeros_1nt)
    norms = _norm_kernel(degp_do, degp_di)
    ew2 = _ew2_kernel(norms, src3, dst3, ew3)

    src_f = src3.reshape(TOTCH, C)
    dw_f = jnp.concatenate(
        [dst3.reshape(TOTCH, 1, C),
         jax.lax.bitcast_convert_type(ew2, jnp.int32).reshape(TOTCH, 1, C)],
        axis=1)

    t1 = _mm_kernel(x_pad, W1)
    p1 = _spmm_kernel(t1, src_f, dw_f, zeros_nt)
    t2 = _postmm_kernel(p1, b1, W2)
    p2 = _spmm_kernel(t2, src_f, dw_f, zeros_nt)
    t3 = _postmm_kernel(p2, b2, W3)
    p3 = _spmm_kernel(t3, src_f, dw_f, zeros_nt)
    return _final_kernel(p3, b3, Wd1, bd1, Wd2, bd2, Wd3, bd3)


# speedup vs baseline: 1.2746x; 1.0573x over previous
import functools

import jax
import jax.numpy as jnp
from jax import lax
from jax.experimental import pallas as pl
from jax.experimental.pallas import tpu as pltpu
from jax.experimental.pallas import tpu_sc as plsc

N = 10000
E = 320000
D = 128

NT = 10112
TILES = 32
CH = 80
C = 128
EP = TILES * CH * C
RPT = NT // 16

_MESH = dict(core_axis_name="c", subcore_axis_name="s", num_cores=2,
             num_subcores=16)
_SC_PARAMS = pltpu.CompilerParams(needs_layout_passes=False)


def _tile_id():
    return lax.axis_index("c") * 16 + lax.axis_index("s")


def _deg_body(src_hbm, dst_hbm, zeros_hbm, out_do, out_di,
              src_v, dst_v, ho_v, hi_v):
    t = _tile_id()
    pltpu.sync_copy(src_hbm.at[t], src_v)
    pltpu.sync_copy(dst_hbm.at[t], dst_v)
    pltpu.sync_copy(zeros_hbm, ho_v)
    pltpu.sync_copy(zeros_hbm, hi_v)
    ones = jnp.ones((16,), jnp.float32)

    def grp(i, _):
        j = i // 8
        m = (i % 8) * 16
        plsc.addupdate_scatter(ho_v, [src_v[j, pl.ds(m, 16)]], ones)
        plsc.addupdate_scatter(hi_v, [dst_v[j, pl.ds(m, 16)]], ones)
        return 0
    lax.fori_loop(0, CH * 8, grp, 0)
    pltpu.sync_copy(ho_v, out_do.at[t])
    pltpu.sync_copy(hi_v, out_di.at[t])


_deg_kernel = functools.partial(
    pl.kernel,
    out_type=[jax.ShapeDtypeStruct((TILES, NT), jnp.float32),
              jax.ShapeDtypeStruct((TILES, NT), jnp.float32)],
    mesh=plsc.VectorSubcoreMesh(**_MESH),
    compiler_params=_SC_PARAMS,
    scratch_types=[
        pltpu.VMEM((CH, C), jnp.int32),
        pltpu.VMEM((CH, C), jnp.int32),
        pltpu.VMEM((NT,), jnp.float32),
        pltpu.VMEM((NT,), jnp.float32),
    ],
)(_deg_body)


def _norm_body(do_ref, di_ref, out_ref):
    deg = jnp.stack([jnp.sum(do_ref[...], axis=0),
                     jnp.sum(di_ref[...], axis=0)])
    out_ref[...] = lax.rsqrt(jnp.maximum(deg, 1.0))


def _norm_kernel(degp_do, degp_di):
    return pl.pallas_call(
        _norm_body,
        out_shape=jax.ShapeDtypeStruct((2, NT), jnp.float32),
    )(degp_do, degp_di)


def _ew2_body(norms_hbm, src_hbm, dst_hbm, ew_hbm, out_hbm,
              no_v, ni_v, src_v, dst_v, ew_v):
    t = _tile_id()
    pltpu.sync_copy(norms_hbm.at[0], no_v)
    pltpu.sync_copy(norms_hbm.at[1], ni_v)
    pltpu.sync_copy(src_hbm.at[t], src_v)
    pltpu.sync_copy(dst_hbm.at[t], dst_v)
    pltpu.sync_copy(ew_hbm.at[t], ew_v)

    def group(i, _):
        j = i // 8
        m = (i % 8) * 16
        sidx = src_v[j, pl.ds(m, 16)]
        didx = dst_v[j, pl.ds(m, 16)]
        a = plsc.load_gather(no_v, [sidx])
        b = plsc.load_gather(ni_v, [didx])
        ew_v[j, pl.ds(m, 16)] = ew_v[j, pl.ds(m, 16)] * a * b
        return 0
    lax.fori_loop(0, CH * 8, group, 0)
    pltpu.sync_copy(ew_v, out_hbm.at[t])


_ew2_kernel = functools.partial(
    pl.kernel,
    out_type=jax.ShapeDtypeStruct((TILES, CH, C), jnp.float32),
    mesh=plsc.VectorSubcoreMesh(**_MESH),
    compiler_params=_SC_PARAMS,
    scratch_types=[
        pltpu.VMEM((NT,), jnp.float32),
        pltpu.VMEM((NT,), jnp.float32),
        pltpu.VMEM((CH, C), jnp.int32),
        pltpu.VMEM((CH, C), jnp.int32),
        pltpu.VMEM((CH, C), jnp.float32),
    ],
)(_ew2_body)


NBUF = 2
CH0 = 128
CH1 = (2 * CH) - CH0
CHMAX = max(CH0, CH1)
TOTCH = TILES * CH


def _spmm_body(h_hbm, src_hbm, dw_hbm, zeros_hbm, out_hbm,
               src_v, dw_v, rows, dwsems, gsems, ssems, sh_agg):
    c = lax.axis_index("c")
    s = lax.axis_index("s")
    nch = jnp.where(c == 0, CH0, CH1)
    base = jnp.where(c == 0, s * CH0, 16 * CH0 + s * CH1)
    nrnd = nch // NBUF
    pltpu.sync_copy(src_hbm.at[pl.ds(base, CHMAX)], src_v)
    pltpu.sync_copy(zeros_hbm, rows.at[0])
    for m in range(RPT // C):
        pltpu.sync_copy(rows.at[0], sh_agg.at[pl.ds(s * RPT + m * C, C)])
    rem = RPT % C
    if rem:
        pltpu.sync_copy(rows.at[0, pl.ds(0, rem)],
                        sh_agg.at[pl.ds(s * RPT + (RPT // C) * C, rem)])

    for b in range(NBUF):
        pltpu.async_copy(dw_hbm.at[base + b], dw_v.at[b], dwsems.at[b])
        pltpu.async_copy(h_hbm.at[src_v.at[b]], rows.at[b], gsems.at[b])
    plsc.subcore_barrier()

    def scale(b):
        def edge(i, _):
            wi = plsc.load_gather(
                dw_v, [jnp.full((16,), b, jnp.int32),
                       jnp.full((16,), 1, jnp.int32),
                       jnp.full((16,), i, jnp.int32)])
            w = plsc.bitcast(wi, jnp.float32)
            for k in range(8):
                rows[b, i, pl.ds(k * 16, 16)] = rows[b, i, pl.ds(k * 16, 16)] * w
            return 0
        lax.fori_loop(0, C, edge, 0, unroll=4)

    def rnd(r, _):
        for b in range(NBUF):
            j = r * NBUF + b
            pltpu.make_async_copy(h_hbm.at[src_v.at[j]], rows.at[b],
                                  gsems.at[b]).wait()
            pltpu.make_async_copy(dw_hbm.at[base + j], dw_v.at[b],
                                  dwsems.at[b]).wait()
            scale(b)
            pltpu.async_copy(rows.at[b], sh_agg.at[dw_v.at[b, 0]],
                             ssems.at[b], add=True)
        for b in range(NBUF):
            j = r * NBUF + b

            @pl.when(r < nrnd - 1)
            def _():
                pltpu.make_async_copy(rows.at[b], sh_agg.at[dw_v.at[b, 0]],
                                      ssems.at[b]).wait()
                pltpu.async_copy(dw_hbm.at[base + j + NBUF], dw_v.at[b],
                                 dwsems.at[b])
                pltpu.async_copy(h_hbm.at[src_v.at[j + NBUF]], rows.at[b],
                                 gsems.at[b])
        return 0
    lax.fori_loop(0, nrnd, rnd, 0)
    for b in range(NBUF):
        pltpu.make_async_copy(rows.at[b], sh_agg.at[dw_v.at[b, 0]],
                              ssems.at[b]).wait()
    plsc.subcore_barrier()
    pltpu.sync_copy(sh_agg.at[pl.ds(s * RPT, RPT)],
                    out_hbm.at[c, pl.ds(s * RPT, RPT)])


_spmm_kernel = functools.partial(
    pl.kernel,
    out_type=jax.ShapeDtypeStruct((2, NT, D), jnp.float32),
    mesh=plsc.VectorSubcoreMesh(**_MESH),
    compiler_params=_SC_PARAMS,
    scratch_types=[
        pltpu.VMEM((CHMAX, C), jnp.int32),
        pltpu.VMEM((NBUF, 2, C), jnp.int32),
        pltpu.VMEM((NBUF, C, D), jnp.float32),
        pltpu.SemaphoreType.DMA((NBUF,)),
        pltpu.SemaphoreType.DMA((NBUF,)),
        pltpu.SemaphoreType.DMA((NBUF,)),
        pltpu.VMEM_SHARED((NT, D), jnp.float32),
    ],
)(_spmm_body)


_RB = 2528


def _mm_body(x_ref, w_ref, o_ref):
    o_ref[...] = jnp.dot(x_ref[...], w_ref[...],
                         preferred_element_type=jnp.float32)


def _mm_kernel(x, w):
    return pl.pallas_call(
        _mm_body,
        grid=(NT // _RB,),
        in_specs=[
            pl.BlockSpec((_RB, D), lambda i: (i, 0)),
            pl.BlockSpec((D, D), lambda i: (0, 0)),
        ],
        out_specs=pl.BlockSpec((_RB, D), lambda i: (i, 0)),
        out_shape=jax.ShapeDtypeStruct((NT, D), jnp.float32),
    )(x, w)


def _postmm_body(p_ref, b_ref, w_ref, o_ref):
    y = jax.nn.relu(p_ref[0] + p_ref[1] + b_ref[...][None, :])
    o_ref[...] = jnp.dot(y, w_ref[...], preferred_element_type=jnp.float32)


def _postmm_kernel(p, b, w):
    return pl.pallas_call(
        _postmm_body,
        grid=(NT // _RB,),
        in_specs=[
            pl.BlockSpec((2, _RB, D), lambda i: (0, i, 0)),
            pl.BlockSpec((D,), lambda i: (0,)),
            pl.BlockSpec((D, D), lambda i: (0, 0)),
        ],
        out_specs=pl.BlockSpec((_RB, D), lambda i: (i, 0)),
        out_shape=jax.ShapeDtypeStruct((NT, D), jnp.float32),
    )(p, b, w)


def _final_body(p_ref, b_ref, wd1_ref, bd1_ref, wd2_ref, bd2_ref,
                wd3_ref, bd3_ref, o_ref):
    y = jax.nn.relu(p_ref[0, :N, :] + p_ref[1, :N, :] + b_ref[...][None, :])
    hg = jnp.sum(y, axis=0, keepdims=True) * (1.0 / N)
    hg = jax.nn.relu(jnp.dot(hg, wd1_ref[...],
                             preferred_element_type=jnp.float32)
                     + bd1_ref[...][None, :])
    hg = jax.nn.relu(jnp.dot(hg, wd2_ref[...],
                             preferred_element_type=jnp.float32)
                     + bd2_ref[...][None, :])
    hg = (jnp.dot(hg, wd3_ref[...], preferred_element_type=jnp.float32)
          + bd3_ref[...][None, :])
    o_ref[...] = jax.nn.sigmoid(hg)


def _final_kernel(p, b3, wd1, bd1, wd2, bd2, wd3, bd3):
    return pl.pallas_call(
        _final_body,
        out_shape=jax.ShapeDtypeStruct((1, 1), jnp.float32),
    )(p, b3, wd1, bd1, wd2, bd2, wd3, bd3)


def kernel(in_feat, edge_index, e_weight, W1, b1, W2, b2, W3, b3,
           Wd1, bd1, Wd2, bd2, Wd3, bd3):
    pad = EP - E
    src3 = jnp.pad(edge_index[0].astype(jnp.int32), (0, pad),
                   constant_values=N).reshape(TILES, CH, C)
    dst3 = jnp.pad(edge_index[1].astype(jnp.int32), (0, pad),
                   constant_values=N).reshape(TILES, CH, C)
    ew3 = jnp.pad(e_weight, (0, pad)).reshape(TILES, CH, C)
    x_pad = jnp.pad(in_feat, ((0, NT - N), (0, 0)))
    zeros_cd = jnp.zeros((C, D), jnp.float32)

    zeros_1nt = jnp.zeros((NT,), jnp.float32)
    degp_do, degp_di = _deg_kernel(src3, dst3, zeros_1nt)
    norms = _norm_kernel(degp_do, degp_di)
    ew2 = _ew2_kernel(norms, src3, dst3, ew3)

    src_f = src3.reshape(TOTCH, C)
    dw_f = jnp.concatenate(
        [dst3.reshape(TOTCH, 1, C),
         jax.lax.bitcast_convert_type(ew2, jnp.int32).reshape(TOTCH, 1, C)],
        axis=1)

    t1 = _mm_kernel(x_pad, W1)
    p1 = _spmm_kernel(t1, src_f, dw_f, zeros_cd)
    t2 = _postmm_kernel(p1, b1, W2)
    p2 = _spmm_kernel(t2, src_f, dw_f, zeros_cd)
    t3 = _postmm_kernel(p2, b2, W3)
    p3 = _spmm_kernel(t3, src_f, dw_f, zeros_cd)
    return _final_kernel(p3, b3, Wd1, bd1, Wd2, bd2, Wd3, bd3)


# speedup vs baseline: 1.3562x; 1.0640x over previous
import functools

import jax
import jax.numpy as jnp
from jax import lax
from jax.experimental import pallas as pl
from jax.experimental.pallas import tpu as pltpu
from jax.experimental.pallas import tpu_sc as plsc

N = 10000
E = 320000
D = 128

NT = 10112
TILES = 32
CH = 80
C = 128
EP = TILES * CH * C
RPT = NT // 16

_MESH = dict(core_axis_name="c", subcore_axis_name="s", num_cores=2,
             num_subcores=16)
_SC_PARAMS = pltpu.CompilerParams(needs_layout_passes=False)


def _tile_id():
    return lax.axis_index("c") * 16 + lax.axis_index("s")


def _deg_body(src_hbm, dst_hbm, zeros_hbm, out_do, out_di,
              src_v, dst_v, ho_v, hi_v):
    t = _tile_id()
    pltpu.sync_copy(src_hbm.at[t], src_v)
    pltpu.sync_copy(dst_hbm.at[t], dst_v)
    pltpu.sync_copy(zeros_hbm, ho_v)
    pltpu.sync_copy(zeros_hbm, hi_v)
    ones = jnp.ones((16,), jnp.float32)

    def grp(i, _):
        j = i // 8
        m = (i % 8) * 16
        plsc.addupdate_scatter(ho_v, [src_v[j, pl.ds(m, 16)]], ones)
        plsc.addupdate_scatter(hi_v, [dst_v[j, pl.ds(m, 16)]], ones)
        return 0
    lax.fori_loop(0, CH * 8, grp, 0)
    pltpu.sync_copy(ho_v, out_do.at[t])
    pltpu.sync_copy(hi_v, out_di.at[t])


_deg_kernel = functools.partial(
    pl.kernel,
    out_type=[jax.ShapeDtypeStruct((TILES, NT), jnp.float32),
              jax.ShapeDtypeStruct((TILES, NT), jnp.float32)],
    mesh=plsc.VectorSubcoreMesh(**_MESH),
    compiler_params=_SC_PARAMS,
    scratch_types=[
        pltpu.VMEM((CH, C), jnp.int32),
        pltpu.VMEM((CH, C), jnp.int32),
        pltpu.VMEM((NT,), jnp.float32),
        pltpu.VMEM((NT,), jnp.float32),
    ],
)(_deg_body)


def _norm_body(do_ref, di_ref, out_ref):
    deg = jnp.stack([jnp.sum(do_ref[...], axis=0),
                     jnp.sum(di_ref[...], axis=0)])
    out_ref[...] = lax.rsqrt(jnp.maximum(deg, 1.0))


def _norm_kernel(degp_do, degp_di):
    return pl.pallas_call(
        _norm_body,
        out_shape=jax.ShapeDtypeStruct((2, NT), jnp.float32),
    )(degp_do, degp_di)


def _ew2_body(norms_hbm, src_hbm, dst_hbm, ew_hbm, out_hbm,
              no_v, ni_v, src_v, dst_v, ew_v):
    t = _tile_id()
    pltpu.sync_copy(norms_hbm.at[0], no_v)
    pltpu.sync_copy(norms_hbm.at[1], ni_v)
    pltpu.sync_copy(src_hbm.at[t], src_v)
    pltpu.sync_copy(dst_hbm.at[t], dst_v)
    pltpu.sync_copy(ew_hbm.at[t], ew_v)

    def group(i, _):
        j = i // 8
        m = (i % 8) * 16
        sidx = src_v[j, pl.ds(m, 16)]
        didx = dst_v[j, pl.ds(m, 16)]
        a = plsc.load_gather(no_v, [sidx])
        b = plsc.load_gather(ni_v, [didx])
        ew_v[j, pl.ds(m, 16)] = ew_v[j, pl.ds(m, 16)] * a * b
        return 0
    lax.fori_loop(0, CH * 8, group, 0)
    pltpu.sync_copy(ew_v, out_hbm.at[t])


_ew2_kernel = functools.partial(
    pl.kernel,
    out_type=jax.ShapeDtypeStruct((TILES, CH, C), jnp.float32),
    mesh=plsc.VectorSubcoreMesh(**_MESH),
    compiler_params=_SC_PARAMS,
    scratch_types=[
        pltpu.VMEM((NT,), jnp.float32),
        pltpu.VMEM((NT,), jnp.float32),
        pltpu.VMEM((CH, C), jnp.int32),
        pltpu.VMEM((CH, C), jnp.int32),
        pltpu.VMEM((CH, C), jnp.float32),
    ],
)(_ew2_body)


NBUF = 2
NSL = 2 * NBUF
CH0 = 144
CH1 = (2 * CH) - CH0
TOTCH = TILES * CH


def _spmm_body(h_hbm, sdw_hbm, zeros_hbm, out_hbm,
               sdw_v, rows, dwsems, gsems, ssems, sh_agg):
    c = lax.axis_index("c")
    s = lax.axis_index("s")
    nch = jnp.where(c == 0, CH0, CH1)
    base = jnp.where(c == 0, s * CH0, 16 * CH0 + s * CH1)
    nrr = nch // NSL
    pltpu.sync_copy(zeros_hbm, rows.at[0])
    for m in range(RPT // C):
        pltpu.sync_copy(rows.at[0], sh_agg.at[pl.ds(s * RPT + m * C, C)])
    rem = RPT % C
    if rem:
        pltpu.sync_copy(rows.at[0, pl.ds(0, rem)],
                        sh_agg.at[pl.ds(s * RPT + (RPT // C) * C, rem)])

    for slot in range(NSL):
        pltpu.async_copy(sdw_hbm.at[base + slot],
                         sdw_v.at[pl.ds(slot * 3, 3)], dwsems.at[slot])
    for slot in range(NBUF):
        pltpu.make_async_copy(sdw_hbm.at[base + slot],
                              sdw_v.at[pl.ds(slot * 3, 3)],
                              dwsems.at[slot]).wait()
    for b in range(NBUF):
        pltpu.async_copy(h_hbm.at[sdw_v.at[b * 3]], rows.at[b], gsems.at[b])
    plsc.subcore_barrier()

    def scale(slot):
        b = slot % NBUF

        def edge(i, _):
            wi = plsc.load_gather(
                sdw_v, [jnp.full((16,), slot * 3 + 2, jnp.int32),
                        jnp.full((16,), i, jnp.int32)])
            w = plsc.bitcast(wi, jnp.float32)
            for k in range(8):
                rows[b, i, pl.ds(k * 16, 16)] = (
                    rows[b, i, pl.ds(k * 16, 16)] * w)
            return 0
        lax.fori_loop(0, C, edge, 0, unroll=4)

    def rnd2(rr, _):
        for q in range(2):
            r = rr * 2 + q
            for b in range(NBUF):
                slot = q * NBUF + b
                j = r * NBUF + b
                pltpu.make_async_copy(h_hbm.at[sdw_v.at[slot * 3]],
                                      rows.at[b], gsems.at[b]).wait()
                scale(slot)
                pltpu.async_copy(rows.at[b], sh_agg.at[sdw_v.at[slot * 3 + 1]],
                                 ssems.at[b], add=True)
            for b in range(NBUF):
                slot = q * NBUF + b
                oslot = (1 - q) * NBUF + b
                j = r * NBUF + b

                @pl.when(j + NBUF < nch)
                def _():
                    pltpu.make_async_copy(rows.at[b],
                                          sh_agg.at[sdw_v.at[slot * 3 + 1]],
                                          ssems.at[b]).wait()
                    pltpu.make_async_copy(sdw_hbm.at[base + j + NBUF],
                                          sdw_v.at[pl.ds(oslot * 3, 3)],
                                          dwsems.at[oslot]).wait()
                    pltpu.async_copy(h_hbm.at[sdw_v.at[oslot * 3]],
                                     rows.at[b], gsems.at[b])

                @pl.when(j + NSL < nch)
                def _():
                    pltpu.async_copy(sdw_hbm.at[base + j + NSL],
                                     sdw_v.at[pl.ds(slot * 3, 3)],
                                     dwsems.at[slot])
        return 0
    lax.fori_loop(0, nrr, rnd2, 0)
    for b in range(NBUF):
        pltpu.make_async_copy(rows.at[b],
                              sh_agg.at[sdw_v.at[(NBUF + b) * 3 + 1]],
                              ssems.at[b]).wait()
    plsc.subcore_barrier()
    pltpu.sync_copy(sh_agg.at[pl.ds(s * RPT, RPT)],
                    out_hbm.at[c, pl.ds(s * RPT, RPT)])


_spmm_kernel = functools.partial(
    pl.kernel,
    out_type=jax.ShapeDtypeStruct((2, NT, D), jnp.float32),
    mesh=plsc.VectorSubcoreMesh(**_MESH),
    compiler_params=_SC_PARAMS,
    scratch_types=[
        pltpu.VMEM((NSL * 3, C), jnp.int32),
        pltpu.VMEM((NBUF, C, D), jnp.float32),
        pltpu.SemaphoreType.DMA((NSL,)),
        pltpu.SemaphoreType.DMA((NBUF,)),
        pltpu.SemaphoreType.DMA((NBUF,)),
        pltpu.VMEM_SHARED((NT, D), jnp.float32),
    ],
)(_spmm_body)


_RB = 2528


def _mm_body(x_ref, w_ref, o_ref):
    o_ref[...] = jnp.dot(x_ref[...], w_ref[...],
                         preferred_element_type=jnp.float32)


def _mm_kernel(x, w):
    return pl.pallas_call(
        _mm_body,
        grid=(NT // _RB,),
        in_specs=[
            pl.BlockSpec((_RB, D), lambda i: (i, 0)),
            pl.BlockSpec((D, D), lambda i: (0, 0)),
        ],
        out_specs=pl.BlockSpec((_RB, D), lambda i: (i, 0)),
        out_shape=jax.ShapeDtypeStruct((NT, D), jnp.float32),
    )(x, w)


def _postmm_body(p_ref, b_ref, w_ref, o_ref):
    y = jax.nn.relu(p_ref[0] + p_ref[1] + b_ref[...][None, :])
    o_ref[...] = jnp.dot(y, w_ref[...], preferred_element_type=jnp.float32)


def _postmm_kernel(p, b, w):
    return pl.pallas_call(
        _postmm_body,
        grid=(NT // _RB,),
        in_specs=[
            pl.BlockSpec((2, _RB, D), lambda i: (0, i, 0)),
            pl.BlockSpec((D,), lambda i: (0,)),
            pl.BlockSpec((D, D), lambda i: (0, 0)),
        ],
        out_specs=pl.BlockSpec((_RB, D), lambda i: (i, 0)),
        out_shape=jax.ShapeDtypeStruct((NT, D), jnp.float32),
    )(p, b, w)


def _final_body(p_ref, b_ref, wd1_ref, bd1_ref, wd2_ref, bd2_ref,
                wd3_ref, bd3_ref, o_ref):
    y = jax.nn.relu(p_ref[0, :N, :] + p_ref[1, :N, :] + b_ref[...][None, :])
    hg = jnp.sum(y, axis=0, keepdims=True) * (1.0 / N)
    hg = jax.nn.relu(jnp.dot(hg, wd1_ref[...],
                             preferred_element_type=jnp.float32)
                     + bd1_ref[...][None, :])
    hg = jax.nn.relu(jnp.dot(hg, wd2_ref[...],
                             preferred_element_type=jnp.float32)
                     + bd2_ref[...][None, :])
    hg = (jnp.dot(hg, wd3_ref[...], preferred_element_type=jnp.float32)
          + bd3_ref[...][None, :])
    o_ref[...] = jax.nn.sigmoid(hg)


def _final_kernel(p, b3, wd1, bd1, wd2, bd2, wd3, bd3):
    return pl.pallas_call(
        _final_body,
        out_shape=jax.ShapeDtypeStruct((1, 1), jnp.float32),
    )(p, b3, wd1, bd1, wd2, bd2, wd3, bd3)


def kernel(in_feat, edge_index, e_weight, W1, b1, W2, b2, W3, b3,
           Wd1, bd1, Wd2, bd2, Wd3, bd3):
    pad = EP - E
    src3 = jnp.pad(edge_index[0].astype(jnp.int32), (0, pad),
                   constant_values=N).reshape(TILES, CH, C)
    dst3 = jnp.pad(edge_index[1].astype(jnp.int32), (0, pad),
                   constant_values=N).reshape(TILES, CH, C)
    ew3 = jnp.pad(e_weight, (0, pad)).reshape(TILES, CH, C)
    x_pad = jnp.pad(in_feat, ((0, NT - N), (0, 0)))
    zeros_cd = jnp.zeros((C, D), jnp.float32)

    zeros_1nt = jnp.zeros((NT,), jnp.float32)
    degp_do, degp_di = _deg_kernel(src3, dst3, zeros_1nt)
    norms = _norm_kernel(degp_do, degp_di)
    ew2 = _ew2_kernel(norms, src3, dst3, ew3)

    sdw_f = jnp.concatenate(
        [src3.reshape(TOTCH, 1, C),
         dst3.reshape(TOTCH, 1, C),
         jax.lax.bitcast_convert_type(ew2, jnp.int32).reshape(TOTCH, 1, C)],
        axis=1)

    t1 = _mm_kernel(x_pad, W1)
    p1 = _spmm_kernel(t1, sdw_f, zeros_cd)
    t2 = _postmm_kernel(p1, b1, W2)
    p2 = _spmm_kernel(t2, sdw_f, zeros_cd)
    t3 = _postmm_kernel(p2, b2, W3)
    p3 = _spmm_kernel(t3, sdw_f, zeros_cd)
    return _final_kernel(p3, b3, Wd1, bd1, Wd2, bd2, Wd3, bd3)
